# baseline (speedup 1.0000x reference)
import jax
import jax.numpy as jnp
from jax.experimental import pallas as pl
from jax.experimental.pallas import tpu as pltpu

E = 8
B = 32
T = 100
P = 2048
N = 1024


def _linear_kernel(st_ref, pm_ref, w_ref, b_ref, x_ref, o_ref,
                   xbuf, obuf, in_sem, out_sem):
    e = pl.program_id(0)
    lo = st_ref[e]
    hi = st_ref[e + 1]
    cnt = hi - lo

    def in_copy(j, slot):
        return pltpu.make_async_copy(
            x_ref.at[pm_ref[j]], xbuf.at[slot], in_sem.at[slot])

    def out_copy(j, slot):
        return pltpu.make_async_copy(
            obuf.at[slot], o_ref.at[pm_ref[j]], out_sem.at[slot])

    @pl.when(cnt > 0)
    def _():
        in_copy(lo, 0).start()

    def body(j, carry):
        slot = jax.lax.rem(j - lo, 2)
        in_copy(j, slot).wait()

        @pl.when(j + 1 < hi)
        def _():
            in_copy(j + 1, 1 - slot).start()

        @pl.when(j - 2 >= lo)
        def _():
            out_copy(j - 2, slot).wait()

        acc = jax.lax.dot_general(
            xbuf[slot].astype(jnp.bfloat16), w_ref[0].astype(jnp.bfloat16),
            dimension_numbers=(((1,), (1,)), ((), ())),
            preferred_element_type=jnp.float32,
        )
        obuf[slot] = acc + b_ref[0]
        out_copy(j, slot).start()
        return carry

    jax.lax.fori_loop(lo, hi, body, 0)

    @pl.when(cnt >= 2)
    def _():
        out_copy(hi - 2, jax.lax.rem(hi - 2 - lo, 2)).wait()

    @pl.when(cnt >= 1)
    def _():
        out_copy(hi - 1, jax.lax.rem(hi - 1 - lo, 2)).wait()


def kernel(x, eid, W, b):
    x = x.reshape(B, T, P)
    iota = jnp.arange(B, dtype=jnp.int32)
    lt = (eid[None, :] < eid[:, None]) | (
        (eid[None, :] == eid[:, None]) & (iota[None, :] < iota[:, None])
    )
    rank = jnp.sum(lt.astype(jnp.int32), axis=1)
    onehot = (rank[None, :] == iota[:, None]).astype(jnp.int32)
    perm = onehot @ iota
    cnt = jnp.sum((eid[None, :] == jnp.arange(E, dtype=jnp.int32)[:, None])
                  .astype(jnp.int32), axis=1)
    start = jnp.concatenate(
        [jnp.zeros((1,), jnp.int32), jnp.cumsum(cnt, dtype=jnp.int32)])
    b3 = b.reshape(E, 1, N)

    grid_spec = pltpu.PrefetchScalarGridSpec(
        num_scalar_prefetch=2,
        grid=(E,),
        in_specs=[
            pl.BlockSpec((1, N, P), lambda e, st, pm: (e, 0, 0)),
            pl.BlockSpec((1, 1, N), lambda e, st, pm: (e, 0, 0)),
            pl.BlockSpec(memory_space=pl.ANY),
        ],
        out_specs=pl.BlockSpec(memory_space=pl.ANY),
        scratch_shapes=[
            pltpu.VMEM((2, T, P), jnp.float32),
            pltpu.VMEM((2, T, N), jnp.float32),
            pltpu.SemaphoreType.DMA((2,)),
            pltpu.SemaphoreType.DMA((2,)),
        ],
    )
    out = pl.pallas_call(
        _linear_kernel,
        grid_spec=grid_spec,
        out_shape=jax.ShapeDtypeStruct((B, T, N), jnp.float32),
    )(start, perm, W, b3, x)
    return out
